# Initial kernel scaffold; baseline (speedup 1.0000x reference)
import jax
import jax.numpy as jnp
from jax.experimental import pallas as pl


def kernel(block_features, block_edge_index, block_edge_attr, ln_in_g, ln_in_b, W_in, b_in, W_e, b_e, Wl1, bl1, Wr1, br1, We1, att1, bo1, ln1_g, ln1_b, Wl2, bl2, Wr2, br2, We2, att2, bo2, ln2_g, ln2_b, Wo1, bo1w, Wo2, bo2w):
    raise NotImplementedError("write your pallas kernel here")


# speedup vs baseline: 5.7452x; 5.7452x over previous
import functools

import jax
import jax.numpy as jnp
from jax import lax
from jax.experimental import pallas as pl
from jax.experimental.pallas import tpu as pltpu
from jax.experimental.pallas import tpu_sc as plsc

_N = 10000
_E = 160000
_E2 = 2 * _E
_D = 512
_H = 4
_C = 128
_NC, _NS, _LN = 2, 16, 16

_RB = 1000
_EB = 2000
_BA = 16
_BB = 32


def _pre_body(bf, g, b, w, bias, o):
    x = bf[...]
    mu = jnp.mean(x, axis=1, keepdims=True)
    v = jnp.mean((x - mu) ** 2, axis=1, keepdims=True)
    xn = (x - mu) / jnp.sqrt(v + 1e-5) * g[...] + b[...]
    y = jnp.dot(xn, w[...], preferred_element_type=jnp.float32) + bias[...]
    o[...] = jnp.maximum(y, 0.0)


def _pre(bf, g, b, w, bias):
    grid = (_N // _RB,)
    return pl.pallas_call(
        _pre_body,
        grid=grid,
        in_specs=[
            pl.BlockSpec((_RB, 256), lambda i: (i, 0)),
            pl.BlockSpec((1, 256), lambda i: (0, 0)),
            pl.BlockSpec((1, 256), lambda i: (0, 0)),
            pl.BlockSpec((256, _D), lambda i: (0, 0)),
            pl.BlockSpec((1, _D), lambda i: (0, 0)),
        ],
        out_specs=pl.BlockSpec((_RB, _D), lambda i: (i, 0)),
        out_shape=jax.ShapeDtypeStruct((_N, _D), jnp.float32),
    )(bf, g, b, w, bias)


def _edge_body(ea, we, be, w1, w2, o1, o2):
    e = jnp.dot(ea[...], we[...], preferred_element_type=jnp.float32) + be[...]
    e = jnp.maximum(e, 0.0)
    o1[...] = jnp.dot(e, w1[...], preferred_element_type=jnp.float32)
    o2[...] = jnp.dot(e, w2[...], preferred_element_type=jnp.float32)


def _edge(ea_pad, we_pad, be, we1, we2):
    grid = (_E // _EB,)
    return pl.pallas_call(
        _edge_body,
        grid=grid,
        in_specs=[
            pl.BlockSpec((_EB, 128), lambda i: (i, 0)),
            pl.BlockSpec((128, _D), lambda i: (0, 0)),
            pl.BlockSpec((1, _D), lambda i: (0, 0)),
            pl.BlockSpec((_D, _D), lambda i: (0, 0)),
            pl.BlockSpec((_D, _D), lambda i: (0, 0)),
        ],
        out_specs=[
            pl.BlockSpec((_EB, _D), lambda i: (i, 0)),
            pl.BlockSpec((_EB, _D), lambda i: (i, 0)),
        ],
        out_shape=[
            jax.ShapeDtypeStruct((_E, _D), jnp.float32),
            jax.ShapeDtypeStruct((_E, _D), jnp.float32),
        ],
    )(ea_pad, we_pad, be, we1, we2)


def _xlr_body(x, wl, bl, wr, br, xlf, xrf, xlh):
    xv = x[...]
    xl = jnp.dot(xv, wl[...], preferred_element_type=jnp.float32) + bl[...]
    xr = jnp.dot(xv, wr[...], preferred_element_type=jnp.float32) + br[...]
    xlf[...] = xl
    xrf[...] = xr
    for h in range(_H):
        xlh[h] = xl[:, h * _C:(h + 1) * _C]


def _xlr(x, wl, bl, wr, br):
    grid = (_N // _RB,)
    return pl.pallas_call(
        _xlr_body,
        grid=grid,
        in_specs=[
            pl.BlockSpec((_RB, _D), lambda i: (i, 0)),
            pl.BlockSpec((_D, _D), lambda i: (0, 0)),
            pl.BlockSpec((1, _D), lambda i: (0, 0)),
            pl.BlockSpec((_D, _D), lambda i: (0, 0)),
            pl.BlockSpec((1, _D), lambda i: (0, 0)),
        ],
        out_specs=[
            pl.BlockSpec((_RB, _D), lambda i: (i, 0)),
            pl.BlockSpec((_RB, _D), lambda i: (i, 0)),
            pl.BlockSpec((_H, _RB, _C), lambda i: (0, i, 0)),
        ],
        out_shape=[
            jax.ShapeDtypeStruct((_N, _D), jnp.float32),
            jax.ShapeDtypeStruct((_N, _D), jnp.float32),
            jax.ShapeDtypeStruct((_H, _N, _C), jnp.float32),
        ],
    )(x, wl, bl, wr, br)


def _post_body(xin, numh, denp, bo, g, b, o):
    den = denp[0] + denp[1]
    parts = []
    for h in range(_H):
        d = den[:, h:h + 1] + 1e-16
        parts.append(numh[h] / d)
    y = jnp.concatenate(parts, axis=1) + bo[...] + xin[...]
    mu = jnp.mean(y, axis=1, keepdims=True)
    v = jnp.mean((y - mu) ** 2, axis=1, keepdims=True)
    o[...] = (y - mu) / jnp.sqrt(v + 1e-5) * g[...] + b[...]


def _post(xin, numh, denp, bo, g, b):
    grid = (_N // _RB,)
    return pl.pallas_call(
        _post_body,
        grid=grid,
        in_specs=[
            pl.BlockSpec((_RB, _D), lambda i: (i, 0)),
            pl.BlockSpec((_H, _RB, _C), lambda i: (0, i, 0)),
            pl.BlockSpec((2, _RB, _C), lambda i: (0, i, 0)),
            pl.BlockSpec((1, _D), lambda i: (0, 0)),
            pl.BlockSpec((1, _D), lambda i: (0, 0)),
            pl.BlockSpec((1, _D), lambda i: (0, 0)),
        ],
        out_specs=pl.BlockSpec((_RB, _D), lambda i: (i, 0)),
        out_shape=jax.ShapeDtypeStruct((_N, _D), jnp.float32),
    )(xin, numh, denp, bo, g, b)


def _out_body(x, w1, b1, w2, b2, o):
    h = jnp.dot(x[...], w1[...], preferred_element_type=jnp.float32) + b1[...]
    h = jnp.maximum(h, 0.0)
    o[...] = jnp.dot(h, w2[...], preferred_element_type=jnp.float32) + b2[...]


def _out(x, w1, b1, w2, b2):
    grid = (_N // _RB,)
    return pl.pallas_call(
        _out_body,
        grid=grid,
        in_specs=[
            pl.BlockSpec((_RB, _D), lambda i: (i, 0)),
            pl.BlockSpec((_D, 256), lambda i: (0, 0)),
            pl.BlockSpec((1, 256), lambda i: (0, 0)),
            pl.BlockSpec((256, 256), lambda i: (0, 0)),
            pl.BlockSpec((1, 256), lambda i: (0, 0)),
        ],
        out_specs=pl.BlockSpec((_RB, 256), lambda i: (i, 0)),
        out_shape=jax.ShapeDtypeStruct((_N, 256), jnp.float32),
    )(x, w1, b1, w2, b2)


def _sc_mesh():
    return plsc.VectorSubcoreMesh(
        core_axis_name="c", subcore_axis_name="s",
        num_cores=_NC, num_subcores=_NS)


def _sca_body(xl_h, xr_h, ep_h, src_h, dst_h, att_h,
              ex_h, denp_h,
              att_v, sidx, didx, xlr, xrr, epr, exb, exb2, zb, den_sh, sem):
    c = lax.axis_index("c")
    s = lax.axis_index("s")
    wid = c * _NS + s
    def zb_zero(i, _):
        for j in range(8):
            zb[i, pl.ds(j * 16, 16)] = jnp.zeros((16,), jnp.float32)
        return _
    lax.fori_loop(0, 25, zb_zero, None)
    for i in range(25):
        pltpu.sync_copy(zb, den_sh.at[pl.ds(s * 625 + i * 25, 25)])
    def exb2_zero(e, _):
        for j in range(8):
            exb2[e, pl.ds(j * 16, 16)] = jnp.zeros((16,), jnp.float32)
        return _
    lax.fori_loop(0, _BA, exb2_zero, None)
    pltpu.sync_copy(att_h, att_v)
    plsc.subcore_barrier()

    ep_off = wid * (_E2 // 32) - c * _E

    def chunk(g, _):
        base = wid * (_E2 // 32) + g * _BA
        pltpu.sync_copy(src_h.at[pl.ds(base, _BA)], sidx)
        pltpu.sync_copy(dst_h.at[pl.ds(base, _BA)], didx)
        pltpu.async_copy(xl_h.at[sidx], xlr, sem).wait()
        pltpu.async_copy(xr_h.at[didx], xrr, sem).wait()
        pltpu.sync_copy(ep_h.at[pl.ds(ep_off + g * _BA, _BA)], epr)

        lane = lax.iota(jnp.int32, 16)
        perms = [lane ^ k for k in (8, 4, 2, 1)]

        def allsum(v):
            for p in perms:
                v = v + jnp.take_along_axis(v, p, axis=0,
                                            mode="promise_in_bounds")
            return v

        def edge(e, _):
            av = jnp.zeros((16,), jnp.float32)
            for h in range(_H):
                acc = jnp.zeros((16,), jnp.float32)
                for j in range(8):
                    o = h * _C + j * 16
                    v = (xlr[e, pl.ds(o, 16)] + xrr[e, pl.ds(o, 16)]
                         + epr[e, pl.ds(o, 16)])
                    m = jnp.maximum(v, 0.2 * v)
                    acc = acc + m * att_v[h, pl.ds(j * 16, 16)]
                av = jnp.where(lane == h, allsum(acc), av)
            ex = jnp.exp(av)
            exb[e] = ex
            exb2[e, pl.ds(0, 16)] = ex
            return _
        lax.fori_loop(0, _BA, edge, None)

        pltpu.sync_copy(exb2, den_sh.at[didx], add=True)
        pltpu.sync_copy(exb, ex_h.at[pl.ds(base, _BA)])
        return _
    lax.fori_loop(0, (_E2 // 32) // _BA, chunk, None)

    plsc.subcore_barrier()
    @pl.when(s == 0)
    def _():
        pltpu.sync_copy(den_sh, denp_h.at[c])


def _sc_pass_a(xl, xr, ep, src, dst, att):
    kfn = pl.kernel(
        _sca_body,
        out_type=(
            jax.ShapeDtypeStruct((_E2, 16), jnp.float32),
            jax.ShapeDtypeStruct((_NC, _N, _C), jnp.float32),
        ),
        mesh=_sc_mesh(),
        scratch_types=[
            pltpu.VMEM((_H, _C), jnp.float32),
            pltpu.VMEM((_BA,), jnp.int32),
            pltpu.VMEM((_BA,), jnp.int32),
            pltpu.VMEM((_BA, _D), jnp.float32),
            pltpu.VMEM((_BA, _D), jnp.float32),
            pltpu.VMEM((_BA, _D), jnp.float32),
            pltpu.VMEM((_BA, 16), jnp.float32),
            pltpu.VMEM((_BA, _C), jnp.float32),
            pltpu.VMEM((25, _C), jnp.float32),
            pltpu.VMEM_SHARED((_N, _C), jnp.float32),
            pltpu.SemaphoreType.DMA,
        ],
    )
    return kfn(xl, xr, ep, src, dst, att)


def _scb_body(xlh_h, ex_h, src_h, dst_h,
              num_h,
              sidx, didx, exv, rows, outv, zb, num_sh, sem):
    c = lax.axis_index("c")
    s = lax.axis_index("s")

    def zb_zero(i, _):
        for j in range(8):
            zb[i, pl.ds(j * 16, 16)] = jnp.zeros((16,), jnp.float32)
        return _
    lax.fori_loop(0, 125, zb_zero, None)

    def run(h):
        def go():
            def chunk(g, _):
                base = s * (_E2 // _NS) + g * _BB
                pltpu.sync_copy(src_h.at[pl.ds(base, _BB)], sidx)
                for t in range(_BB // 16):
                    sidx[pl.ds(t * 16, 16)] = sidx[pl.ds(t * 16, 16)] + h * _N
                pltpu.async_copy(xlh_h.at[sidx], rows, sem).wait()
                pltpu.sync_copy(ex_h.at[pl.ds(base, _BB)], exv)
                pltpu.sync_copy(dst_h.at[pl.ds(base, _BB)], didx)

                def edge(e, _):
                    a = jnp.full((16,), exv[e][h])
                    for j in range(8):
                        outv[e, pl.ds(j * 16, 16)] = (
                            rows[e, pl.ds(j * 16, 16)] * a)
                    return _
                lax.fori_loop(0, _BB, edge, None)

                pltpu.sync_copy(outv, num_sh.at[didx], add=True)
                return _
            lax.fori_loop(0, (_E2 // _NS) // _BB, chunk, None)
        return go

    for hp in range(2):
        for i in range(5):
            pltpu.sync_copy(zb, num_sh.at[pl.ds(s * 625 + i * 125, 125)])
        plsc.subcore_barrier()

        lax.cond(c == 0, run(hp), run(2 + hp))

        plsc.subcore_barrier()
        @pl.when(s == 0)
        def _():
            pltpu.sync_copy(num_sh, num_h.at[c * 2 + hp])
        plsc.subcore_barrier()


def _sc_pass_b(xlh_flat, ex, src, dst):
    kfn = pl.kernel(
        _scb_body,
        out_type=jax.ShapeDtypeStruct((_H, _N, _C), jnp.float32),
        mesh=_sc_mesh(),
        scratch_types=[
            pltpu.VMEM((_BB,), jnp.int32),
            pltpu.VMEM((_BB,), jnp.int32),
            pltpu.VMEM((_BB, 16), jnp.float32),
            pltpu.VMEM((_BB, _C), jnp.float32),
            pltpu.VMEM((_BB, _C), jnp.float32),
            pltpu.VMEM((125, _C), jnp.float32),
            pltpu.VMEM_SHARED((_N, _C), jnp.float32),
            pltpu.SemaphoreType.DMA,
        ],
    )
    return kfn(xlh_flat, ex, src, dst)


def kernel(block_features, block_edge_index, block_edge_attr,
           ln_in_g, ln_in_b, W_in, b_in, W_e, b_e,
           Wl1, bl1, Wr1, br1, We1, att1, bo1, ln1_g, ln1_b,
           Wl2, bl2, Wr2, br2, We2, att2, bo2, ln2_g, ln2_b,
           Wo1, bo1w, Wo2, bo2w):
    r2 = lambda v: v.reshape(1, -1)
    src = jnp.concatenate([block_edge_index[0], block_edge_index[1]]).astype(jnp.int32)
    dst = jnp.concatenate([block_edge_index[1], block_edge_index[0]]).astype(jnp.int32)

    ea_pad = jnp.pad(block_edge_attr, ((0, 0), (0, 128 - 16)))
    we_pad = jnp.pad(W_e, ((0, 128 - 16), (0, 0)))

    x = _pre(block_features, r2(ln_in_g), r2(ln_in_b), W_in, r2(b_in))
    ep1, ep2 = _edge(ea_pad, we_pad, r2(b_e), We1, We2)

    def layer(xin, ep, Wl, bl, Wr, br, att, bo, g, b):
        xlf, xrf, xlh = _xlr(xin, Wl, r2(bl), Wr, r2(br))
        ex, denp = _sc_pass_a(xlf, xrf, ep, src, dst, att)
        numh = _sc_pass_b(xlh.reshape(_H * _N, _C), ex, src, dst)
        return _post(xin, numh, denp, r2(bo), r2(g), r2(b))

    x = layer(x, ep1, Wl1, bl1, Wr1, br1, att1, bo1, ln1_g, ln1_b)
    x = layer(x, ep2, Wl2, bl2, Wr2, br2, att2, bo2, ln2_g, ln2_b)
    return _out(x, Wo1, r2(bo1w), Wo2, r2(bo2w))


# speedup vs baseline: 11.8592x; 2.0642x over previous
import functools

import jax
import jax.numpy as jnp
from jax import lax
from jax.experimental import pallas as pl
from jax.experimental.pallas import tpu as pltpu
from jax.experimental.pallas import tpu_sc as plsc

_N = 10000
_E = 160000
_E2 = 2 * _E
_D = 512
_H = 4
_C = 128
_NC, _NS, _LN = 2, 16, 16

_RB = 1000
_EB = 2000
_BA = 80
_EPT = _E2 // _NS


def _pre_body(bf, g, b, w, bias, o):
    x = bf[...]
    mu = jnp.mean(x, axis=1, keepdims=True)
    v = jnp.mean((x - mu) ** 2, axis=1, keepdims=True)
    xn = (x - mu) / jnp.sqrt(v + 1e-5) * g[...] + b[...]
    y = jnp.dot(xn, w[...], preferred_element_type=jnp.float32) + bias[...]
    o[...] = jnp.maximum(y, 0.0)


def _pre(bf, g, b, w, bias):
    grid = (_N // _RB,)
    return pl.pallas_call(
        _pre_body,
        grid=grid,
        in_specs=[
            pl.BlockSpec((_RB, 256), lambda i: (i, 0)),
            pl.BlockSpec((1, 256), lambda i: (0, 0)),
            pl.BlockSpec((1, 256), lambda i: (0, 0)),
            pl.BlockSpec((256, _D), lambda i: (0, 0)),
            pl.BlockSpec((1, _D), lambda i: (0, 0)),
        ],
        out_specs=pl.BlockSpec((_RB, _D), lambda i: (i, 0)),
        out_shape=jax.ShapeDtypeStruct((_N, _D), jnp.float32),
    )(bf, g, b, w, bias)


def _edge_body(ea, we, be, w1, w2, o1, o2):
    e = jnp.dot(ea[...], we[...], preferred_element_type=jnp.float32) + be[...]
    e = jnp.maximum(e, 0.0)
    ep1 = jnp.dot(e, w1[...], preferred_element_type=jnp.float32)
    ep2 = jnp.dot(e, w2[...], preferred_element_type=jnp.float32)
    for h in range(_H):
        o1[h] = ep1[:, h * _C:(h + 1) * _C]
        o2[h] = ep2[:, h * _C:(h + 1) * _C]


def _edge(ea_pad, we_pad, be, we1, we2):
    grid = (_E // _EB,)
    return pl.pallas_call(
        _edge_body,
        grid=grid,
        in_specs=[
            pl.BlockSpec((_EB, 128), lambda i: (i, 0)),
            pl.BlockSpec((128, _D), lambda i: (0, 0)),
            pl.BlockSpec((1, _D), lambda i: (0, 0)),
            pl.BlockSpec((_D, _D), lambda i: (0, 0)),
            pl.BlockSpec((_D, _D), lambda i: (0, 0)),
        ],
        out_specs=[
            pl.BlockSpec((_H, _EB, _C), lambda i: (0, i, 0)),
            pl.BlockSpec((_H, _EB, _C), lambda i: (0, i, 0)),
        ],
        out_shape=[
            jax.ShapeDtypeStruct((_H, _E, _C), jnp.float32),
            jax.ShapeDtypeStruct((_H, _E, _C), jnp.float32),
        ],
    )(ea_pad, we_pad, be, we1, we2)


def _xlr_body(x, wl, bl, wr, br, xlh, xrh):
    xv = x[...]
    xl = jnp.dot(xv, wl[...], preferred_element_type=jnp.float32) + bl[...]
    xr = jnp.dot(xv, wr[...], preferred_element_type=jnp.float32) + br[...]
    for h in range(_H):
        xlh[h] = xl[:, h * _C:(h + 1) * _C]
        xrh[h] = xr[:, h * _C:(h + 1) * _C]


def _xlr(x, wl, bl, wr, br):
    grid = (_N // _RB,)
    return pl.pallas_call(
        _xlr_body,
        grid=grid,
        in_specs=[
            pl.BlockSpec((_RB, _D), lambda i: (i, 0)),
            pl.BlockSpec((_D, _D), lambda i: (0, 0)),
            pl.BlockSpec((1, _D), lambda i: (0, 0)),
            pl.BlockSpec((_D, _D), lambda i: (0, 0)),
            pl.BlockSpec((1, _D), lambda i: (0, 0)),
        ],
        out_specs=[
            pl.BlockSpec((_H, _RB, _C), lambda i: (0, i, 0)),
            pl.BlockSpec((_H, _RB, _C), lambda i: (0, i, 0)),
        ],
        out_shape=[
            jax.ShapeDtypeStruct((_H, _N, _C), jnp.float32),
            jax.ShapeDtypeStruct((_H, _N, _C), jnp.float32),
        ],
    )(x, wl, bl, wr, br)


def _post_body(xin, numh, bo, g, b, o):
    y = jnp.concatenate([numh[h] for h in range(_H)], axis=1)
    y = y + bo[...] + xin[...]
    mu = jnp.mean(y, axis=1, keepdims=True)
    v = jnp.mean((y - mu) ** 2, axis=1, keepdims=True)
    o[...] = (y - mu) / jnp.sqrt(v + 1e-5) * g[...] + b[...]


def _post(xin, numh, bo, g, b):
    grid = (_N // _RB,)
    return pl.pallas_call(
        _post_body,
        grid=grid,
        in_specs=[
            pl.BlockSpec((_RB, _D), lambda i: (i, 0)),
            pl.BlockSpec((_H, _RB, _C), lambda i: (0, i, 0)),
            pl.BlockSpec((1, _D), lambda i: (0, 0)),
            pl.BlockSpec((1, _D), lambda i: (0, 0)),
            pl.BlockSpec((1, _D), lambda i: (0, 0)),
        ],
        out_specs=pl.BlockSpec((_RB, _D), lambda i: (i, 0)),
        out_shape=jax.ShapeDtypeStruct((_N, _D), jnp.float32),
    )(xin, numh, bo, g, b)


def _out_body(x, w1, b1, w2, b2, o):
    h = jnp.dot(x[...], w1[...], preferred_element_type=jnp.float32) + b1[...]
    h = jnp.maximum(h, 0.0)
    o[...] = jnp.dot(h, w2[...], preferred_element_type=jnp.float32) + b2[...]


def _out(x, w1, b1, w2, b2):
    grid = (_N // _RB,)
    return pl.pallas_call(
        _out_body,
        grid=grid,
        in_specs=[
            pl.BlockSpec((_RB, _D), lambda i: (i, 0)),
            pl.BlockSpec((_D, 256), lambda i: (0, 0)),
            pl.BlockSpec((1, 256), lambda i: (0, 0)),
            pl.BlockSpec((256, 256), lambda i: (0, 0)),
            pl.BlockSpec((1, 256), lambda i: (0, 0)),
        ],
        out_specs=pl.BlockSpec((_RB, 256), lambda i: (i, 0)),
        out_shape=jax.ShapeDtypeStruct((_N, 256), jnp.float32),
    )(x, w1, b1, w2, b2)


def _sce_body(xlh_h, xrh_h, eph_h, src_h, dst_h, att_h,
              num_out,
              att_row, sidx, didxh, didx, xa, xb, epr, exb,
              numv, denv, num_sh, den_sh, s1, s2, s3):
    c = lax.axis_index("c")
    s = lax.axis_index("s")
    lane = lax.iota(jnp.int32, 16)
    perms = [lane ^ k for k in (8, 4, 2, 1)]

    def allsum(v):
        for p in perms:
            v = v + jnp.take_along_axis(v, p, axis=0,
                                        mode="promise_in_bounds")
        return v

    for hp in range(2):
        h = c * 2 + hp

        def numv_zero(i, _):
            for j in range(8):
                numv[i, pl.ds(j * 16, 16)] = jnp.zeros((16,), jnp.float32)
            denv[i] = jnp.zeros((16,), jnp.float32)
            return _
        lax.fori_loop(0, 25, numv_zero, None)
        for i in range(25):
            pltpu.sync_copy(numv, num_sh.at[pl.ds(s * 625 + i * 25, 25)])
            pltpu.sync_copy(denv, den_sh.at[pl.ds(s * 625 + i * 25, 25)])
        pltpu.sync_copy(att_h.at[h], att_row)
        plsc.subcore_barrier()

        att_js = [att_row[pl.ds(j * 16, 16)] for j in range(8)]
        hn = h * _N
        he = h * _E
        tile_base = s * _EPT
        ep_tile = jnp.where(tile_base >= _E, tile_base - _E, tile_base) + he

        def chunk(g, _):
            base = tile_base + g * _BA
            pltpu.sync_copy(src_h.at[pl.ds(base, _BA)], sidx)
            pltpu.sync_copy(dst_h.at[pl.ds(base, _BA)], didx)
            for t in range(_BA // 16):
                sl = pl.ds(t * 16, 16)
                sidx[sl] = sidx[sl] + hn
                didxh[sl] = didx[sl] + hn
            cp1 = pltpu.async_copy(xlh_h.at[sidx], xa, s1)
            cp2 = pltpu.async_copy(xrh_h.at[didxh], xb, s2)
            cp3 = pltpu.async_copy(eph_h.at[pl.ds(ep_tile + g * _BA, _BA)],
                                   epr, s3)
            cp1.wait()
            cp2.wait()
            cp3.wait()

            def edge(e, _):
                xaj = [xa[e, pl.ds(j * 16, 16)] for j in range(8)]
                acc = jnp.zeros((16,), jnp.float32)
                for j in range(8):
                    sl = pl.ds(j * 16, 16)
                    v = xaj[j] + xb[e, sl] + epr[e, sl]
                    m = jnp.maximum(v, 0.2 * v)
                    acc = acc + m * att_js[j]
                ex = jnp.exp(allsum(acc))
                exb[e] = ex
                for j in range(8):
                    xa[e, pl.ds(j * 16, 16)] = xaj[j] * ex
                return _
            lax.fori_loop(0, _BA, edge, None, unroll=2)

            pltpu.sync_copy(xa, num_sh.at[didx], add=True)
            pltpu.sync_copy(exb, den_sh.at[didx], add=True)
            return _
        lax.fori_loop(0, _EPT // _BA, chunk, None)

        plsc.subcore_barrier()

        for i in range(25):
            rows = s * 625 + i * 25
            pltpu.sync_copy(num_sh.at[pl.ds(rows, 25)], numv)
            pltpu.sync_copy(den_sh.at[pl.ds(rows, 25)], denv)

            def norm(r, _):
                d = denv[r] + 1e-16
                for j in range(8):
                    sl = pl.ds(j * 16, 16)
                    numv[r, sl] = numv[r, sl] / d
                return _
            lax.fori_loop(0, 25, norm, None)
            pltpu.sync_copy(numv, num_out.at[h, pl.ds(rows, 25)])
        plsc.subcore_barrier()


def _sc_edge(xlh_flat, xrh_flat, eph_flat, src, dst, att):
    kfn = pl.kernel(
        _sce_body,
        out_type=jax.ShapeDtypeStruct((_H, _N, _C), jnp.float32),
        mesh=plsc.VectorSubcoreMesh(
            core_axis_name="c", subcore_axis_name="s",
            num_cores=_NC, num_subcores=_NS),
        compiler_params=pltpu.CompilerParams(use_tc_tiling_on_sc=False),
        scratch_types=[
            pltpu.VMEM((_C,), jnp.float32),
            pltpu.VMEM((_BA,), jnp.int32),
            pltpu.VMEM((_BA,), jnp.int32),
            pltpu.VMEM((_BA,), jnp.int32),
            pltpu.VMEM((_BA, _C), jnp.float32),
            pltpu.VMEM((_BA, _C), jnp.float32),
            pltpu.VMEM((_BA, _C), jnp.float32),
            pltpu.VMEM((_BA, 16), jnp.float32),
            pltpu.VMEM((25, _C), jnp.float32),
            pltpu.VMEM((25, 16), jnp.float32),
            pltpu.VMEM_SHARED((_N, _C), jnp.float32),
            pltpu.VMEM_SHARED((_N, 16), jnp.float32),
            pltpu.SemaphoreType.DMA,
            pltpu.SemaphoreType.DMA,
            pltpu.SemaphoreType.DMA,
        ],
    )
    return kfn(xlh_flat, xrh_flat, eph_flat, src, dst, att)


def kernel(block_features, block_edge_index, block_edge_attr,
           ln_in_g, ln_in_b, W_in, b_in, W_e, b_e,
           Wl1, bl1, Wr1, br1, We1, att1, bo1, ln1_g, ln1_b,
           Wl2, bl2, Wr2, br2, We2, att2, bo2, ln2_g, ln2_b,
           Wo1, bo1w, Wo2, bo2w):
    r2 = lambda v: v.reshape(1, -1)
    src = jnp.concatenate([block_edge_index[0], block_edge_index[1]]).astype(jnp.int32)
    dst = jnp.concatenate([block_edge_index[1], block_edge_index[0]]).astype(jnp.int32)

    ea_pad = jnp.pad(block_edge_attr, ((0, 0), (0, 128 - 16)))
    we_pad = jnp.pad(W_e, ((0, 128 - 16), (0, 0)))

    x = _pre(block_features, r2(ln_in_g), r2(ln_in_b), W_in, r2(b_in))
    eph1, eph2 = _edge(ea_pad, we_pad, r2(b_e), We1, We2)

    def layer(xin, eph, Wl, bl, Wr, br, att, bo, g, b):
        xlh, xrh = _xlr(xin, Wl, r2(bl), Wr, r2(br))
        numh = _sc_edge(xlh.reshape(_H * _N, _C), xrh.reshape(_H * _N, _C),
                        eph.reshape(_H * _E, _C), src, dst, att)
        return _post(xin, numh, r2(bo), r2(g), r2(b))

    x = layer(x, eph1, Wl1, bl1, Wr1, br1, att1, bo1, ln1_g, ln1_b)
    x = layer(x, eph2, Wl2, bl2, Wr2, br2, att2, bo2, ln2_g, ln2_b)
    return _out(x, Wo1, r2(bo1w), Wo2, r2(bo2w))


# speedup vs baseline: 11.8681x; 1.0008x over previous
import functools

import jax
import jax.numpy as jnp
from jax import lax
from jax.experimental import pallas as pl
from jax.experimental.pallas import tpu as pltpu
from jax.experimental.pallas import tpu_sc as plsc

_N = 10000
_E = 160000
_E2 = 2 * _E
_D = 512
_H = 4
_C = 128
_NC, _NS, _LN = 2, 16, 16

_RB = 1000
_EB = 2000
_BA = 80
_EPT = _E2 // _NS


def _pre_body(bf, g, b, w, bias, o):
    x = bf[...]
    mu = jnp.mean(x, axis=1, keepdims=True)
    v = jnp.mean((x - mu) ** 2, axis=1, keepdims=True)
    xn = (x - mu) / jnp.sqrt(v + 1e-5) * g[...] + b[...]
    y = jnp.dot(xn, w[...], preferred_element_type=jnp.float32) + bias[...]
    o[...] = jnp.maximum(y, 0.0)


def _pre(bf, g, b, w, bias):
    grid = (_N // _RB,)
    return pl.pallas_call(
        _pre_body,
        grid=grid,
        in_specs=[
            pl.BlockSpec((_RB, 256), lambda i: (i, 0)),
            pl.BlockSpec((1, 256), lambda i: (0, 0)),
            pl.BlockSpec((1, 256), lambda i: (0, 0)),
            pl.BlockSpec((256, _D), lambda i: (0, 0)),
            pl.BlockSpec((1, _D), lambda i: (0, 0)),
        ],
        out_specs=pl.BlockSpec((_RB, _D), lambda i: (i, 0)),
        out_shape=jax.ShapeDtypeStruct((_N, _D), jnp.float32),
    )(bf, g, b, w, bias)


def _edge_body(ea, we, be, w1, w2, o1, o2):
    e = jnp.dot(ea[...], we[...], preferred_element_type=jnp.float32) + be[...]
    e = jnp.maximum(e, 0.0)
    ep1 = jnp.dot(e, w1[...], preferred_element_type=jnp.float32)
    ep2 = jnp.dot(e, w2[...], preferred_element_type=jnp.float32)
    for h in range(_H):
        o1[h] = ep1[:, h * _C:(h + 1) * _C]
        o2[h] = ep2[:, h * _C:(h + 1) * _C]


def _edge(ea_pad, we_pad, be, we1, we2):
    grid = (_E // _EB,)
    return pl.pallas_call(
        _edge_body,
        grid=grid,
        in_specs=[
            pl.BlockSpec((_EB, 128), lambda i: (i, 0)),
            pl.BlockSpec((128, _D), lambda i: (0, 0)),
            pl.BlockSpec((1, _D), lambda i: (0, 0)),
            pl.BlockSpec((_D, _D), lambda i: (0, 0)),
            pl.BlockSpec((_D, _D), lambda i: (0, 0)),
        ],
        out_specs=[
            pl.BlockSpec((_H, _EB, _C), lambda i: (0, i, 0)),
            pl.BlockSpec((_H, _EB, _C), lambda i: (0, i, 0)),
        ],
        out_shape=[
            jax.ShapeDtypeStruct((_H, _E, _C), jnp.float32),
            jax.ShapeDtypeStruct((_H, _E, _C), jnp.float32),
        ],
    )(ea_pad, we_pad, be, we1, we2)


def _xlr_body(x, wl, bl, wr, br, xlh, xrh):
    xv = x[...]
    xl = jnp.dot(xv, wl[...], preferred_element_type=jnp.float32) + bl[...]
    xr = jnp.dot(xv, wr[...], preferred_element_type=jnp.float32) + br[...]
    for h in range(_H):
        xlh[h] = xl[:, h * _C:(h + 1) * _C]
        xrh[h] = xr[:, h * _C:(h + 1) * _C]


def _xlr(x, wl, bl, wr, br):
    grid = (_N // _RB,)
    return pl.pallas_call(
        _xlr_body,
        grid=grid,
        in_specs=[
            pl.BlockSpec((_RB, _D), lambda i: (i, 0)),
            pl.BlockSpec((_D, _D), lambda i: (0, 0)),
            pl.BlockSpec((1, _D), lambda i: (0, 0)),
            pl.BlockSpec((_D, _D), lambda i: (0, 0)),
            pl.BlockSpec((1, _D), lambda i: (0, 0)),
        ],
        out_specs=[
            pl.BlockSpec((_H, _RB, _C), lambda i: (0, i, 0)),
            pl.BlockSpec((_H, _RB, _C), lambda i: (0, i, 0)),
        ],
        out_shape=[
            jax.ShapeDtypeStruct((_H, _N, _C), jnp.float32),
            jax.ShapeDtypeStruct((_H, _N, _C), jnp.float32),
        ],
    )(x, wl, bl, wr, br)


def _post_body(xin, numh, bo, g, b, o):
    y = jnp.concatenate([numh[h] for h in range(_H)], axis=1)
    y = y + bo[...] + xin[...]
    mu = jnp.mean(y, axis=1, keepdims=True)
    v = jnp.mean((y - mu) ** 2, axis=1, keepdims=True)
    o[...] = (y - mu) / jnp.sqrt(v + 1e-5) * g[...] + b[...]


def _post(xin, numh, bo, g, b):
    grid = (_N // _RB,)
    return pl.pallas_call(
        _post_body,
        grid=grid,
        in_specs=[
            pl.BlockSpec((_RB, _D), lambda i: (i, 0)),
            pl.BlockSpec((_H, _RB, _C), lambda i: (0, i, 0)),
            pl.BlockSpec((1, _D), lambda i: (0, 0)),
            pl.BlockSpec((1, _D), lambda i: (0, 0)),
            pl.BlockSpec((1, _D), lambda i: (0, 0)),
        ],
        out_specs=pl.BlockSpec((_RB, _D), lambda i: (i, 0)),
        out_shape=jax.ShapeDtypeStruct((_N, _D), jnp.float32),
    )(xin, numh, bo, g, b)


def _out_body(x, w1, b1, w2, b2, o):
    h = jnp.dot(x[...], w1[...], preferred_element_type=jnp.float32) + b1[...]
    h = jnp.maximum(h, 0.0)
    o[...] = jnp.dot(h, w2[...], preferred_element_type=jnp.float32) + b2[...]


def _out(x, w1, b1, w2, b2):
    grid = (_N // _RB,)
    return pl.pallas_call(
        _out_body,
        grid=grid,
        in_specs=[
            pl.BlockSpec((_RB, _D), lambda i: (i, 0)),
            pl.BlockSpec((_D, 256), lambda i: (0, 0)),
            pl.BlockSpec((1, 256), lambda i: (0, 0)),
            pl.BlockSpec((256, 256), lambda i: (0, 0)),
            pl.BlockSpec((1, 256), lambda i: (0, 0)),
        ],
        out_specs=pl.BlockSpec((_RB, 256), lambda i: (i, 0)),
        out_shape=jax.ShapeDtypeStruct((_N, 256), jnp.float32),
    )(x, w1, b1, w2, b2)


def _sce_body(xlh_h, xrh_h, eph_h, src_h, dst_h, att_h,
              num_out,
              att_row, sidx, didxh, didx, xa, xb, epr, exb,
              numv, denv, num_sh, den_sh, s1, s2, s3):
    c = lax.axis_index("c")
    s = lax.axis_index("s")
    lane = lax.iota(jnp.int32, 16)
    perms = [lane ^ k for k in (8, 4, 2, 1)]

    def allsum(v):
        for p in perms:
            v = v + jnp.take_along_axis(v, p, axis=0,
                                        mode="promise_in_bounds")
        return v

    for hp in range(2):
        h = c * 2 + hp

        def numv_zero(i, _):
            for j in range(8):
                numv[i, pl.ds(j * 16, 16)] = jnp.zeros((16,), jnp.float32)
            denv[i] = jnp.zeros((16,), jnp.float32)
            return _
        lax.fori_loop(0, 25, numv_zero, None)
        for i in range(25):
            pltpu.sync_copy(numv, num_sh.at[pl.ds(s * 625 + i * 25, 25)])
            pltpu.sync_copy(denv, den_sh.at[pl.ds(s * 625 + i * 25, 25)])
        pltpu.sync_copy(att_h.at[h], att_row)
        plsc.subcore_barrier()

        att_js = [att_row[pl.ds(j * 16, 16)] for j in range(8)]
        hn = h * _N
        he = h * _E
        tile_base = s * _EPT
        ep_tile = jnp.where(tile_base >= _E, tile_base - _E, tile_base) + he

        def chunk(g, _):
            base = tile_base + g * _BA
            pltpu.sync_copy(src_h.at[pl.ds(base, _BA)], sidx)
            pltpu.sync_copy(dst_h.at[pl.ds(base, _BA)], didx)
            for t in range(_BA // 16):
                sl = pl.ds(t * 16, 16)
                sidx[sl] = sidx[sl] + hn
                didxh[sl] = didx[sl] + hn
            cp1 = pltpu.async_copy(xlh_h.at[sidx], xa, s1)
            cp2 = pltpu.async_copy(xrh_h.at[didxh], xb, s2)
            cp3 = pltpu.async_copy(eph_h.at[pl.ds(ep_tile + g * _BA, _BA)],
                                   epr, s3)
            cp1.wait()
            cp2.wait()
            cp3.wait()

            def edge(e, _):
                xaj = [xa[e, pl.ds(j * 16, 16)] for j in range(8)]
                acc = jnp.zeros((16,), jnp.float32)
                for j in range(8):
                    sl = pl.ds(j * 16, 16)
                    v = xaj[j] + xb[e, sl] + epr[e, sl]
                    m = jnp.maximum(v, 0.2 * v)
                    acc = acc + m * att_js[j]
                ex = jnp.exp(allsum(acc))
                exb[e] = ex
                for j in range(8):
                    xa[e, pl.ds(j * 16, 16)] = xaj[j] * ex
                return _
            lax.fori_loop(0, _BA, edge, None, unroll=4)

            pltpu.sync_copy(xa, num_sh.at[didx], add=True)
            pltpu.sync_copy(exb, den_sh.at[didx], add=True)
            return _
        lax.fori_loop(0, _EPT // _BA, chunk, None)

        plsc.subcore_barrier()

        for i in range(25):
            rows = s * 625 + i * 25
            pltpu.sync_copy(num_sh.at[pl.ds(rows, 25)], numv)
            pltpu.sync_copy(den_sh.at[pl.ds(rows, 25)], denv)

            def norm(r, _):
                d = denv[r] + 1e-16
                for j in range(8):
                    sl = pl.ds(j * 16, 16)
                    numv[r, sl] = numv[r, sl] / d
                return _
            lax.fori_loop(0, 25, norm, None)
            pltpu.sync_copy(numv, num_out.at[h, pl.ds(rows, 25)])
        plsc.subcore_barrier()


def _sc_edge(xlh_flat, xrh_flat, eph_flat, src, dst, att):
    kfn = pl.kernel(
        _sce_body,
        out_type=jax.ShapeDtypeStruct((_H, _N, _C), jnp.float32),
        mesh=plsc.VectorSubcoreMesh(
            core_axis_name="c", subcore_axis_name="s",
            num_cores=_NC, num_subcores=_NS),
        compiler_params=pltpu.CompilerParams(use_tc_tiling_on_sc=False),
        scratch_types=[
            pltpu.VMEM((_C,), jnp.float32),
            pltpu.VMEM((_BA,), jnp.int32),
            pltpu.VMEM((_BA,), jnp.int32),
            pltpu.VMEM((_BA,), jnp.int32),
            pltpu.VMEM((_BA, _C), jnp.float32),
            pltpu.VMEM((_BA, _C), jnp.float32),
            pltpu.VMEM((_BA, _C), jnp.float32),
            pltpu.VMEM((_BA, 16), jnp.float32),
            pltpu.VMEM((25, _C), jnp.float32),
            pltpu.VMEM((25, 16), jnp.float32),
            pltpu.VMEM_SHARED((_N, _C), jnp.float32),
            pltpu.VMEM_SHARED((_N, 16), jnp.float32),
            pltpu.SemaphoreType.DMA,
            pltpu.SemaphoreType.DMA,
            pltpu.SemaphoreType.DMA,
        ],
    )
    return kfn(xlh_flat, xrh_flat, eph_flat, src, dst, att)


def kernel(block_features, block_edge_index, block_edge_attr,
           ln_in_g, ln_in_b, W_in, b_in, W_e, b_e,
           Wl1, bl1, Wr1, br1, We1, att1, bo1, ln1_g, ln1_b,
           Wl2, bl2, Wr2, br2, We2, att2, bo2, ln2_g, ln2_b,
           Wo1, bo1w, Wo2, bo2w):
    r2 = lambda v: v.reshape(1, -1)
    src = jnp.concatenate([block_edge_index[0], block_edge_index[1]]).astype(jnp.int32)
    dst = jnp.concatenate([block_edge_index[1], block_edge_index[0]]).astype(jnp.int32)

    ea_pad = jnp.pad(block_edge_attr, ((0, 0), (0, 128 - 16)))
    we_pad = jnp.pad(W_e, ((0, 128 - 16), (0, 0)))

    x = _pre(block_features, r2(ln_in_g), r2(ln_in_b), W_in, r2(b_in))
    eph1, eph2 = _edge(ea_pad, we_pad, r2(b_e), We1, We2)

    def layer(xin, eph, Wl, bl, Wr, br, att, bo, g, b):
        xlh, xrh = _xlr(xin, Wl, r2(bl), Wr, r2(br))
        numh = _sc_edge(xlh.reshape(_H * _N, _C), xrh.reshape(_H * _N, _C),
                        eph.reshape(_H * _E, _C), src, dst, att)
        return _post(xin, numh, r2(bo), r2(g), r2(b))

    x = layer(x, eph1, Wl1, bl1, Wr1, br1, att1, bo1, ln1_g, ln1_b)
    x = layer(x, eph2, Wl2, bl2, Wr2, br2, att2, bo2, ln2_g, ln2_b)
    return _out(x, Wo1, r2(bo1w), Wo2, r2(bo2w))


# speedup vs baseline: 15.7788x; 1.3295x over previous
import functools

import jax
import jax.numpy as jnp
from jax import lax
from jax.experimental import pallas as pl
from jax.experimental.pallas import tpu as pltpu
from jax.experimental.pallas import tpu_sc as plsc

_N = 10000
_E = 160000
_E2 = 2 * _E
_D = 512
_H = 4
_C = 128
_NC, _NS, _LN = 2, 16, 16

_RB = 1000
_EB = 2000
_BA = 80
_EPT = _E2 // _NS


def _pre_body(bf, g, b, w, bias, o):
    x = bf[...]
    mu = jnp.mean(x, axis=1, keepdims=True)
    v = jnp.mean((x - mu) ** 2, axis=1, keepdims=True)
    xn = (x - mu) / jnp.sqrt(v + 1e-5) * g[...] + b[...]
    y = jnp.dot(xn, w[...], preferred_element_type=jnp.float32) + bias[...]
    o[...] = jnp.maximum(y, 0.0)


def _pre(bf, g, b, w, bias):
    grid = (_N // _RB,)
    return pl.pallas_call(
        _pre_body,
        grid=grid,
        in_specs=[
            pl.BlockSpec((_RB, 256), lambda i: (i, 0)),
            pl.BlockSpec((1, 256), lambda i: (0, 0)),
            pl.BlockSpec((1, 256), lambda i: (0, 0)),
            pl.BlockSpec((256, _D), lambda i: (0, 0)),
            pl.BlockSpec((1, _D), lambda i: (0, 0)),
        ],
        out_specs=pl.BlockSpec((_RB, _D), lambda i: (i, 0)),
        out_shape=jax.ShapeDtypeStruct((_N, _D), jnp.float32),
    )(bf, g, b, w, bias)


def _edge_body(ea, we, be, w1, w2, o1, o2):
    e = jnp.dot(ea[...], we[...], preferred_element_type=jnp.float32) + be[...]
    e = jnp.maximum(e, 0.0)
    ep1 = jnp.dot(e, w1[...], preferred_element_type=jnp.float32)
    ep2 = jnp.dot(e, w2[...], preferred_element_type=jnp.float32)
    for h in range(_H):
        o1[h] = ep1[:, h * _C:(h + 1) * _C]
        o2[h] = ep2[:, h * _C:(h + 1) * _C]


def _edge(ea_pad, we_pad, be, we1, we2):
    grid = (_E // _EB,)
    return pl.pallas_call(
        _edge_body,
        grid=grid,
        in_specs=[
            pl.BlockSpec((_EB, 128), lambda i: (i, 0)),
            pl.BlockSpec((128, _D), lambda i: (0, 0)),
            pl.BlockSpec((1, _D), lambda i: (0, 0)),
            pl.BlockSpec((_D, _D), lambda i: (0, 0)),
            pl.BlockSpec((_D, _D), lambda i: (0, 0)),
        ],
        out_specs=[
            pl.BlockSpec((_H, _EB, _C), lambda i: (0, i, 0)),
            pl.BlockSpec((_H, _EB, _C), lambda i: (0, i, 0)),
        ],
        out_shape=[
            jax.ShapeDtypeStruct((_H, _E, _C), jnp.float32),
            jax.ShapeDtypeStruct((_H, _E, _C), jnp.float32),
        ],
    )(ea_pad, we_pad, be, we1, we2)


def _xlr_body(x, wl, bl, wr, br, xlh, xrh):
    xv = x[...]
    xl = jnp.dot(xv, wl[...], preferred_element_type=jnp.float32) + bl[...]
    xr = jnp.dot(xv, wr[...], preferred_element_type=jnp.float32) + br[...]
    for h in range(_H):
        xlh[h] = xl[:, h * _C:(h + 1) * _C]
        xrh[h] = xr[:, h * _C:(h + 1) * _C]


def _xlr(x, wl, bl, wr, br):
    grid = (_N // _RB,)
    return pl.pallas_call(
        _xlr_body,
        grid=grid,
        in_specs=[
            pl.BlockSpec((_RB, _D), lambda i: (i, 0)),
            pl.BlockSpec((_D, _D), lambda i: (0, 0)),
            pl.BlockSpec((1, _D), lambda i: (0, 0)),
            pl.BlockSpec((_D, _D), lambda i: (0, 0)),
            pl.BlockSpec((1, _D), lambda i: (0, 0)),
        ],
        out_specs=[
            pl.BlockSpec((_H, _RB, _C), lambda i: (0, i, 0)),
            pl.BlockSpec((_H, _RB, _C), lambda i: (0, i, 0)),
        ],
        out_shape=[
            jax.ShapeDtypeStruct((_H, _N, _C), jnp.float32),
            jax.ShapeDtypeStruct((_H, _N, _C), jnp.float32),
        ],
    )(x, wl, bl, wr, br)


def _post_body(xin, numh, bo, g, b, o):
    y = jnp.concatenate([numh[h] for h in range(_H)], axis=1)
    y = y + bo[...] + xin[...]
    mu = jnp.mean(y, axis=1, keepdims=True)
    v = jnp.mean((y - mu) ** 2, axis=1, keepdims=True)
    o[...] = (y - mu) / jnp.sqrt(v + 1e-5) * g[...] + b[...]


def _post(xin, numh, bo, g, b):
    grid = (_N // _RB,)
    return pl.pallas_call(
        _post_body,
        grid=grid,
        in_specs=[
            pl.BlockSpec((_RB, _D), lambda i: (i, 0)),
            pl.BlockSpec((_H, _RB, _C), lambda i: (0, i, 0)),
            pl.BlockSpec((1, _D), lambda i: (0, 0)),
            pl.BlockSpec((1, _D), lambda i: (0, 0)),
            pl.BlockSpec((1, _D), lambda i: (0, 0)),
        ],
        out_specs=pl.BlockSpec((_RB, _D), lambda i: (i, 0)),
        out_shape=jax.ShapeDtypeStruct((_N, _D), jnp.float32),
    )(xin, numh, bo, g, b)


def _out_body(x, w1, b1, w2, b2, o):
    h = jnp.dot(x[...], w1[...], preferred_element_type=jnp.float32) + b1[...]
    h = jnp.maximum(h, 0.0)
    o[...] = jnp.dot(h, w2[...], preferred_element_type=jnp.float32) + b2[...]


def _out(x, w1, b1, w2, b2):
    grid = (_N // _RB,)
    return pl.pallas_call(
        _out_body,
        grid=grid,
        in_specs=[
            pl.BlockSpec((_RB, _D), lambda i: (i, 0)),
            pl.BlockSpec((_D, 256), lambda i: (0, 0)),
            pl.BlockSpec((1, 256), lambda i: (0, 0)),
            pl.BlockSpec((256, 256), lambda i: (0, 0)),
            pl.BlockSpec((1, 256), lambda i: (0, 0)),
        ],
        out_specs=pl.BlockSpec((_RB, 256), lambda i: (i, 0)),
        out_shape=jax.ShapeDtypeStruct((_N, 256), jnp.float32),
    )(x, w1, b1, w2, b2)


def _sce_body(xlh_h, xrh_h, eph_h, src_h, dst_h, att_h,
              num_out,
              att_row, sidx, didxh, didx, xa, xb, epr, exb,
              numv, denv, num_sh, den_sh, s1, s2, s3):
    c = lax.axis_index("c")
    s = lax.axis_index("s")
    lane = lax.iota(jnp.int32, 16)
    perms = [lane ^ k for k in (8, 4, 2, 1)]

    def allsum(v):
        for p in perms:
            v = v + jnp.take_along_axis(v, p, axis=0,
                                        mode="promise_in_bounds")
        return v

    for hp in range(2):
        h = c * 2 + hp

        def numv_zero(i, _):
            for j in range(8):
                numv[i, pl.ds(j * 16, 16)] = jnp.zeros((16,), jnp.float32)
            denv[i] = jnp.zeros((16,), jnp.float32)
            return _
        lax.fori_loop(0, 25, numv_zero, None)
        for i in range(25):
            pltpu.sync_copy(numv, num_sh.at[pl.ds(s * 625 + i * 25, 25)])
            pltpu.sync_copy(denv, den_sh.at[pl.ds(s * 625 + i * 25, 25)])
        pltpu.sync_copy(att_h.at[h], att_row)
        plsc.subcore_barrier()

        att_js = [att_row[pl.ds(j * 16, 16)] for j in range(8)]
        hn = h * _N
        he = h * _E
        tile_base = s * _EPT
        ep_tile = jnp.where(tile_base >= _E, tile_base - _E, tile_base) + he

        def chunk(g, _):
            base = tile_base + g * _BA
            pltpu.sync_copy(src_h.at[pl.ds(base, _BA)], sidx)
            pltpu.sync_copy(dst_h.at[pl.ds(base, _BA)], didx)
            for t in range(_BA // 16):
                sl = pl.ds(t * 16, 16)
                sidx[sl] = sidx[sl] + hn
                didxh[sl] = didx[sl] + hn
            cp1 = pltpu.async_copy(xlh_h.at[sidx], xa, s1)
            cp2 = pltpu.async_copy(xrh_h.at[didxh], xb, s2)
            cp3 = pltpu.async_copy(eph_h.at[pl.ds(ep_tile + g * _BA, _BA)],
                                   epr, s3)
            cp1.wait()
            cp2.wait()
            cp3.wait()

            @plsc.parallel_loop(0, _BA, 1, unroll=4)
            def edge(e):
                xaj = [xa[e, pl.ds(j * 16, 16)] for j in range(8)]
                acc = jnp.zeros((16,), jnp.float32)
                for j in range(8):
                    sl = pl.ds(j * 16, 16)
                    v = xaj[j] + xb[e, sl] + epr[e, sl]
                    m = jnp.maximum(v, 0.2 * v)
                    acc = acc + m * att_js[j]
                ex = jnp.exp(allsum(acc))
                exb[e] = ex
                for j in range(8):
                    xa[e, pl.ds(j * 16, 16)] = xaj[j] * ex

            pltpu.sync_copy(xa, num_sh.at[didx], add=True)
            pltpu.sync_copy(exb, den_sh.at[didx], add=True)
            return _
        lax.fori_loop(0, _EPT // _BA, chunk, None)

        plsc.subcore_barrier()

        for i in range(25):
            rows = s * 625 + i * 25
            pltpu.sync_copy(num_sh.at[pl.ds(rows, 25)], numv)
            pltpu.sync_copy(den_sh.at[pl.ds(rows, 25)], denv)

            def norm(r, _):
                d = denv[r] + 1e-16
                for j in range(8):
                    sl = pl.ds(j * 16, 16)
                    numv[r, sl] = numv[r, sl] / d
                return _
            lax.fori_loop(0, 25, norm, None)
            pltpu.sync_copy(numv, num_out.at[h, pl.ds(rows, 25)])
        plsc.subcore_barrier()


def _sc_edge(xlh_flat, xrh_flat, eph_flat, src, dst, att):
    kfn = pl.kernel(
        _sce_body,
        out_type=jax.ShapeDtypeStruct((_H, _N, _C), jnp.float32),
        mesh=plsc.VectorSubcoreMesh(
            core_axis_name="c", subcore_axis_name="s",
            num_cores=_NC, num_subcores=_NS),
        compiler_params=pltpu.CompilerParams(use_tc_tiling_on_sc=False),
        scratch_types=[
            pltpu.VMEM((_C,), jnp.float32),
            pltpu.VMEM((_BA,), jnp.int32),
            pltpu.VMEM((_BA,), jnp.int32),
            pltpu.VMEM((_BA,), jnp.int32),
            pltpu.VMEM((_BA, _C), jnp.float32),
            pltpu.VMEM((_BA, _C), jnp.float32),
            pltpu.VMEM((_BA, _C), jnp.float32),
            pltpu.VMEM((_BA, 16), jnp.float32),
            pltpu.VMEM((25, _C), jnp.float32),
            pltpu.VMEM((25, 16), jnp.float32),
            pltpu.VMEM_SHARED((_N, _C), jnp.float32),
            pltpu.VMEM_SHARED((_N, 16), jnp.float32),
            pltpu.SemaphoreType.DMA,
            pltpu.SemaphoreType.DMA,
            pltpu.SemaphoreType.DMA,
        ],
    )
    return kfn(xlh_flat, xrh_flat, eph_flat, src, dst, att)


def kernel(block_features, block_edge_index, block_edge_attr,
           ln_in_g, ln_in_b, W_in, b_in, W_e, b_e,
           Wl1, bl1, Wr1, br1, We1, att1, bo1, ln1_g, ln1_b,
           Wl2, bl2, Wr2, br2, We2, att2, bo2, ln2_g, ln2_b,
           Wo1, bo1w, Wo2, bo2w):
    r2 = lambda v: v.reshape(1, -1)
    src = jnp.concatenate([block_edge_index[0], block_edge_index[1]]).astype(jnp.int32)
    dst = jnp.concatenate([block_edge_index[1], block_edge_index[0]]).astype(jnp.int32)

    ea_pad = jnp.pad(block_edge_attr, ((0, 0), (0, 128 - 16)))
    we_pad = jnp.pad(W_e, ((0, 128 - 16), (0, 0)))

    x = _pre(block_features, r2(ln_in_g), r2(ln_in_b), W_in, r2(b_in))
    eph1, eph2 = _edge(ea_pad, we_pad, r2(b_e), We1, We2)

    def layer(xin, eph, Wl, bl, Wr, br, att, bo, g, b):
        xlh, xrh = _xlr(xin, Wl, r2(bl), Wr, r2(br))
        numh = _sc_edge(xlh.reshape(_H * _N, _C), xrh.reshape(_H * _N, _C),
                        eph.reshape(_H * _E, _C), src, dst, att)
        return _post(xin, numh, r2(bo), r2(g), r2(b))

    x = layer(x, eph1, Wl1, bl1, Wr1, br1, att1, bo1, ln1_g, ln1_b)
    x = layer(x, eph2, Wl2, bl2, Wr2, br2, att2, bo2, ln2_g, ln2_b)
    return _out(x, Wo1, r2(bo1w), Wo2, r2(bo2w))
